# baseline (speedup 1.0000x reference)
import jax
import jax.numpy as jnp
from jax import lax
from jax.experimental import pallas as pl
from jax.experimental.pallas import tpu as pltpu

_BETA = 0.9
_LAM = 3.0
_EPS = 1e-4
_J = 1024


def _body(idx_ref, p_ref, t_ref, tb_ref, o_ref, g_buf, sem):
    i = pl.program_id(0)
    nb = pl.num_programs(0)

    def issue(step, slot):
        for j in range(_J):
            r = idx_ref[step * _J + j]
            pltpu.make_async_copy(
                tb_ref.at[r], g_buf.at[slot, j], sem.at[slot]).start()

    @pl.when(i == 0)
    def _():
        issue(i, 0)

    @pl.when(i + 1 < nb)
    def _():
        issue(i + 1, (i + 1) % 2)

    slot = i % 2
    p = p_ref[...].T
    t = t_ref[0, 0, :]
    m = jnp.max(p, axis=1, keepdims=True)
    e = jnp.exp(p - m)
    z = jnp.sum(e, axis=1, keepdims=True)
    y = jnp.clip(e * (1.0 / z), _EPS, 1.0 - _EPS)
    s1 = jnp.sum(y, axis=1, keepdims=True)
    s2 = jnp.sum(y * y, axis=1, keepdims=True)
    cls = lax.broadcasted_iota(jnp.int32, p.shape, 1)
    pt = jnp.sum(jnp.where(cls == t[:, None], p, 0.0), axis=1,
                 keepdims=True)
    ce = m + jnp.log(z) - pt

    for j in range(_J):
        pltpu.make_async_copy(
            tb_ref.at[0], g_buf.at[slot, j], sem.at[slot]).wait()
    g = g_buf[slot]
    d = jnp.sum(g * y, axis=1, keepdims=True)
    elr = jnp.log(1.0 - (_BETA * d + (1.0 - _BETA) * s2 / s1))
    o_ref[0, 0, :] = (ce + _LAM * elr)[:, 0]


def kernel(predictions, targets, indices, targets_buffer):
    B, C = predictions.shape
    nb = B // _J
    t3 = targets.reshape(nb, 1, _J)

    grid_spec = pltpu.PrefetchScalarGridSpec(
        num_scalar_prefetch=1,
        grid=(nb,),
        in_specs=[
            pl.BlockSpec((C, _J), lambda i, idx: (0, i)),
            pl.BlockSpec((1, 1, _J), lambda i, idx: (i, 0, 0)),
            pl.BlockSpec(memory_space=pl.ANY),
        ],
        out_specs=pl.BlockSpec((1, 1, _J), lambda i, idx: (i, 0, 0)),
        scratch_shapes=[
            pltpu.VMEM((2, _J, C), jnp.float32),
            pltpu.SemaphoreType.DMA((2,)),
        ],
    )
    out = pl.pallas_call(
        _body,
        grid_spec=grid_spec,
        out_shape=jax.ShapeDtypeStruct((nb, 1, _J), jnp.float32),
    )(indices, predictions.T, t3, targets_buffer)
    return out.reshape(B)
